# Initial kernel scaffold; baseline (speedup 1.0000x reference)
import jax
import jax.numpy as jnp
from jax.experimental import pallas as pl


def kernel(x, mp0, mp1, W1, b1, Wg, bg, Wo, bo):
    raise NotImplementedError("write your pallas kernel here")


# speedup vs baseline: 4.5751x; 4.5751x over previous
import functools

import jax
import jax.numpy as jnp
from jax import lax
from jax.experimental import pallas as pl
from jax.experimental.pallas import tpu as pltpu
from jax.experimental.pallas import tpu_sc as plsc

N = 10000
D = 128
H = 128
E = 320000

NUM_CORES = 2
NUM_SUBCORES = 16
EDGES_PER_TILE = E // NUM_SUBCORES
CHUNK = 80
NUM_CHUNKS = EDGES_PER_TILE // CHUNK
N_PAD = 10240
ROWS_PER_TILE = N_PAD // NUM_SUBCORES

BN = 1000


def _support_body(x_ref, w1_ref, b1_ref, wg_ref, out_ref):
    h = jnp.dot(x_ref[...], w1_ref[...], preferred_element_type=jnp.float32)
    h = jnp.maximum(h + b1_ref[...], 0.0)
    out_ref[...] = jnp.dot(h, wg_ref[...], preferred_element_type=jnp.float32)


def _support(x, W1, b1, Wg):
    return pl.pallas_call(
        _support_body,
        grid=(N // BN,),
        in_specs=[
            pl.BlockSpec((BN, D), lambda i: (i, 0)),
            pl.BlockSpec((D, H), lambda i: (0, 0)),
            pl.BlockSpec((1, H), lambda i: (0, 0)),
            pl.BlockSpec((H, H), lambda i: (0, 0)),
        ],
        out_specs=pl.BlockSpec((BN, H), lambda i: (i, 0)),
        out_shape=jax.ShapeDtypeStruct((N, H), jnp.float32),
    )(x, W1, b1.reshape(1, H), Wg)


def _edge_body(support, srcs, dsts, zero, agg, accum, sidx, didx, rows):
    cid = lax.axis_index("c")
    tid = lax.axis_index("s")
    row0 = tid * ROWS_PER_TILE
    pltpu.sync_copy(zero.at[pl.ds(row0, ROWS_PER_TILE)],
                    accum.at[pl.ds(row0, ROWS_PER_TILE)])
    plsc.subcore_barrier()

    e0 = cid * E + tid * EDGES_PER_TILE

    def step(i, carry):
        off = e0 + i * CHUNK
        pltpu.sync_copy(srcs.at[pl.ds(off, CHUNK)], sidx)
        pltpu.sync_copy(dsts.at[pl.ds(off, CHUNK)], didx)
        pltpu.sync_copy(support.at[sidx], rows)
        pltpu.sync_copy(rows, accum.at[didx], add=True)
        return carry

    lax.fori_loop(0, NUM_CHUNKS, step, 0)
    plsc.subcore_barrier()
    pltpu.sync_copy(accum.at[pl.ds(row0, ROWS_PER_TILE)],
                    agg.at[cid, pl.ds(row0, ROWS_PER_TILE)])


@functools.partial(
    pl.kernel,
    out_type=jax.ShapeDtypeStruct((NUM_CORES, N_PAD, H), jnp.float32),
    mesh=plsc.VectorSubcoreMesh(
        core_axis_name="c", subcore_axis_name="s",
        num_cores=NUM_CORES, num_subcores=NUM_SUBCORES),
    scratch_types=[
        pltpu.VMEM_SHARED((N_PAD, H), jnp.float32),
        pltpu.VMEM((CHUNK,), jnp.int32),
        pltpu.VMEM((CHUNK,), jnp.int32),
        pltpu.VMEM((CHUNK, H), jnp.float32),
    ],
)
def _edge_agg(support, srcs, dsts, zero, agg, accum, sidx, didx, rows):
    _edge_body(support, srcs, dsts, zero, agg, accum, sidx, didx, rows)


def _q_body(a0_ref, a1_ref, bg_ref, wo_ref, bo_ref, out_ref):
    ea = jnp.maximum(a0_ref[...] + bg_ref[...], 0.0)
    eb = jnp.maximum(a1_ref[...] + bg_ref[...], 0.0)
    emb = 0.5 * (ea + eb)
    out_ref[...] = (
        jnp.dot(emb, wo_ref[...], preferred_element_type=jnp.float32)
        + bo_ref[...])


def _q_out(agg0, agg1, bg, Wo, bo):
    return pl.pallas_call(
        _q_body,
        grid=(N // BN,),
        in_specs=[
            pl.BlockSpec((BN, H), lambda i: (i, 0)),
            pl.BlockSpec((BN, H), lambda i: (i, 0)),
            pl.BlockSpec((1, H), lambda i: (0, 0)),
            pl.BlockSpec((H, 1), lambda i: (0, 0)),
            pl.BlockSpec((1, 1), lambda i: (0, 0)),
        ],
        out_specs=pl.BlockSpec((BN, 1), lambda i: (i, 0)),
        out_shape=jax.ShapeDtypeStruct((N, 1), jnp.float32),
    )(agg0, agg1, bg.reshape(1, H), Wo, bo.reshape(1, 1))


def kernel(x, mp0, mp1, W1, b1, Wg, bg, Wo, bo):
    support = _support(x, W1, b1, Wg)
    srcs = jnp.concatenate([mp0[0], mp1[0]]).astype(jnp.int32)
    dsts = jnp.concatenate([mp0[1], mp1[1]]).astype(jnp.int32)
    zero = jnp.zeros((N_PAD, H), jnp.float32)
    agg = _edge_agg(support, srcs, dsts, zero)
    return _q_out(agg[0, :N], agg[1, :N], bg, Wo, bo)
